# baseline (speedup 1.0000x reference)
import functools

import jax
import jax.numpy as jnp
from jax import lax
from jax.experimental import pallas as pl
from jax.experimental.pallas import tpu as pltpu
from jax.experimental.pallas import tpu_sc as plsc

_B = 1024
_H = 50
_PAD = 100000
_CW = 128
_NC = 2
_NS = 16
_NW = _NC * _NS
_PERW = _B * _H // _NW
_CH = 100
_NCH = _PERW // _CH
_UPW = _B // _NW


_BM = 4096


def _proj_body(a_ref, itm_ref, w_ref, b_ref, o_ref):
    i = pl.program_id(0)
    t = jnp.tanh(jnp.dot(a_ref[...], w_ref[...], preferred_element_type=jnp.float32)
                 + b_ref[...])
    rid = i * _BM + lax.broadcasted_iota(jnp.int32, (_BM, 1), 0)
    t = jnp.where(rid == _PAD, 0.0, t)
    z = jnp.zeros((_BM, _CW - 80), jnp.float32)
    o_ref[...] = jnp.concatenate([t, itm_ref[...], z], axis=1)


def _build_table(genome, item_table, w_t, bias):
    m, k = genome.shape
    grid = (m + _BM - 1) // _BM
    return pl.pallas_call(
        _proj_body,
        grid=(grid,),
        in_specs=[
            pl.BlockSpec((_BM, k), lambda i: (i, 0)),
            pl.BlockSpec((_BM, 40), lambda i: (i, 0)),
            pl.BlockSpec((k, 40), lambda i: (0, 0)),
            pl.BlockSpec((1, 40), lambda i: (0, 0)),
        ],
        out_specs=pl.BlockSpec((_BM, _CW), lambda i: (i, 0)),
        out_shape=jax.ShapeDtypeStruct((m, _CW), jnp.float32),
    )(genome, item_table, w_t, bias)


def _sc_body(idx_hbm, tgt_hbm, rat_hbm, tab_hbm, outp_hbm, outt_hbm,
             idx_v, wv_v, row_v, out_v, tgt_v, trow_v, sem):
    c = lax.axis_index("c")
    s = lax.axis_index("s")
    w = s * _NC + c
    pltpu.sync_copy(idx_hbm.at[w], idx_v)
    pltpu.sync_copy(rat_hbm.at[w], wv_v)

    def chunk(j, carry):
        pltpu.async_copy(tab_hbm.at[idx_v.at[j]], row_v, sem).wait()
        for u in range(2):
            def hloop(h, accs):
                wvec = wv_v[pl.ds((j * _CH + u * _H + h) * 16, 16)]
                row = u * _H + h
                return tuple(accs[ss] + wvec * row_v[row, pl.ds(16 * ss, 16)]
                             for ss in range(8))

            accs = lax.fori_loop(
                0, _H, hloop,
                tuple(jnp.zeros((16,), jnp.float32) for _ in range(8)))
            for ss in range(8):
                out_v[j * 2 + u, pl.ds(16 * ss, 16)] = accs[ss]
        return carry

    lax.fori_loop(0, _NCH, chunk, 0)
    pltpu.sync_copy(out_v, outp_hbm.at[pl.ds(w * _UPW, _UPW)])

    pltpu.sync_copy(tgt_hbm.at[w], tgt_v)
    pltpu.async_copy(tab_hbm.at[tgt_v], trow_v, sem).wait()
    pltpu.sync_copy(trow_v, outt_hbm.at[pl.ds(w * _UPW, _UPW)])


def _sc_gather_pool(idx3, tgt2, rat2, tab):
    fn = functools.partial(
        pl.kernel,
        out_type=[
            jax.ShapeDtypeStruct((_B, _CW), jnp.float32),
            jax.ShapeDtypeStruct((_B, _CW), jnp.float32),
        ],
        mesh=plsc.VectorSubcoreMesh(core_axis_name="c", subcore_axis_name="s"),
        scratch_types=[
            pltpu.VMEM((_NCH, _CH), jnp.int32),
            pltpu.VMEM((_PERW * 16,), jnp.float32),
            pltpu.VMEM((_CH, _CW), jnp.float32),
            pltpu.VMEM((_UPW, _CW), jnp.float32),
            pltpu.VMEM((_UPW,), jnp.int32),
            pltpu.VMEM((_UPW, _CW), jnp.float32),
            pltpu.SemaphoreType.DMA,
        ],
    )(_sc_body)
    return fn(idx3, tgt2, rat2, tab)


_BB = 128


def _combine_body(p_ref, hidx_ref, rat_ref, ugc_ref, ts_ref, yr_ref,
                  mg_ref, mt_ref, mgt_ref, tg_ref,
                  wug_ref, bug_ref, tst_ref, wts_ref, bts_ref,
                  yrt_ref, wyr_ref, byr_ref,
                  wig_ref, big_ref, wit_ref, bit_ref,
                  wgn_ref, bgn_ref, wie_ref, bie_ref, o_ref):
    f32 = jnp.float32
    wgt = rat_ref[...] * (hidx_ref[...] != _PAD).astype(f32)
    ws = jnp.clip(jnp.sum(jnp.abs(wgt), axis=1, keepdims=True), 1e-6, None)
    pooled = p_ref[...] / ws
    gen = pooled[:, :35]
    hist = pooled[:, 40:80]

    dot = lambda a, b: jnp.dot(a, b, preferred_element_type=f32)
    genre = jnp.tanh(dot(ugc_ref[...], wug_ref[...]) + bug_ref[...])
    ts_oh = (lax.broadcasted_iota(jnp.int32, (_BB, 100), 1) == ts_ref[...]).astype(f32)
    tse = jnp.tanh(dot(dot(ts_oh, tst_ref[...]), wts_ref[...]) + bts_ref[...])
    yr_oh = (lax.broadcasted_iota(jnp.int32, (_BB, 120), 1) == yr_ref[...]).astype(f32)
    yre = jnp.tanh(dot(dot(yr_oh, yrt_ref[...]), wyr_ref[...]) + byr_ref[...])

    ig = jnp.tanh(dot(mg_ref[...], wig_ref[...]) + big_ref[...])
    it = jnp.tanh(dot(mt_ref[...], wit_ref[...]) + bit_ref[...])
    ign = jnp.tanh(dot(mgt_ref[...], wgn_ref[...]) + bgn_ref[...])
    ie = jnp.tanh(dot(tg_ref[...][:, 40:80], wie_ref[...]) + bie_ref[...])

    u = jnp.concatenate([hist, gen, genre, tse], axis=1)
    v = jnp.concatenate([ig, it, ign, ie, yre], axis=1)
    o_ref[...] = jnp.sum(u * v, axis=1, keepdims=True)


def _combine(pooled, hidx, rat, ugc, ts2, yr2, mg, mt, mgt, tgt_rows, consts):
    grid = _B // _BB
    row = lambda i: (i, 0)
    rep = lambda i: (0, 0)
    in_specs = [
        pl.BlockSpec((_BB, _CW), row),
        pl.BlockSpec((_BB, _H), row),
        pl.BlockSpec((_BB, _H), row),
        pl.BlockSpec((_BB, 20), row),
        pl.BlockSpec((_BB, 1), row),
        pl.BlockSpec((_BB, 1), row),
        pl.BlockSpec((_BB, 20), row),
        pl.BlockSpec((_BB, 1000), row),
        pl.BlockSpec((_BB, 1128), row),
        pl.BlockSpec((_BB, _CW), row),
    ] + [pl.BlockSpec(c.shape, rep) for c in consts]
    return pl.pallas_call(
        _combine_body,
        grid=(grid,),
        in_specs=in_specs,
        out_specs=pl.BlockSpec((_BB, 1), row),
        out_shape=jax.ShapeDtypeStruct((_B, 1), jnp.float32),
    )(pooled, hidx, rat, ugc, ts2, yr2, mg, mt, mgt, tgt_rows, *consts)


def kernel(user_genre_contexts, user_watch_history, user_watch_history_ratings,
           timestamps, movie_genres, movie_tags, movie_genome_tags, years,
           target_movieId, genome_context_buffer, item_table, Wie, bie, Wig, big,
           Wit, bit, Wgn, bgn, year_table, Wyr, byr, Wug, bug, ts_table, Wts, bts):
    f32 = jnp.float32
    wgn_t_pad = jnp.zeros((Wgn.shape[1], 40), f32).at[:, :35].set(Wgn.T)
    bgn_pad = jnp.zeros((1, 40), f32).at[0, :35].set(bgn)
    tab = _build_table(genome_context_buffer, item_table, wgn_t_pad, bgn_pad)

    idx = user_watch_history.astype(jnp.int32)
    idx3 = idx.reshape(_NW, _NCH, _CH)
    tgt2 = target_movieId.astype(jnp.int32).reshape(_NW, _UPW)
    rat2 = jnp.broadcast_to(user_watch_history_ratings.reshape(_B * _H, 1),
                            (_B * _H, 16)).reshape(_NW, _PERW * 16)
    pooled, tgtg = _sc_gather_pool(idx3, tgt2, rat2, tab)

    consts = [
        Wug.T, bug.reshape(1, -1), ts_table, Wts.T, bts.reshape(1, -1),
        year_table, Wyr.T, byr.reshape(1, -1),
        Wig.T, big.reshape(1, -1), Wit.T, bit.reshape(1, -1),
        Wgn.T, bgn.reshape(1, -1), Wie.T, bie.reshape(1, -1),
    ]
    out = _combine(pooled, idx, user_watch_history_ratings, user_genre_contexts,
                   timestamps.astype(jnp.int32).reshape(_B, 1),
                   years.astype(jnp.int32).reshape(_B, 1),
                   movie_genres, movie_tags, movie_genome_tags, tgtg, consts)
    return out.reshape(_B)


# speedup vs baseline: 1.0189x; 1.0189x over previous
import functools

import jax
import jax.numpy as jnp
from jax import lax
from jax.experimental import pallas as pl
from jax.experimental.pallas import tpu as pltpu
from jax.experimental.pallas import tpu_sc as plsc

_B = 1024
_H = 50
_PAD = 100000
_CW = 128
_NC = 2
_NS = 16
_NW = _NC * _NS
_PERW = _B * _H // _NW
_CH = 100
_NCH = _PERW // _CH
_UPW = _B // _NW


_BM = 2048


def _proj_body(a_ref, itm_ref, w_ref, b_ref, o_ref):
    i = pl.program_id(0)
    t = jnp.tanh(jnp.dot(a_ref[...], w_ref[...], preferred_element_type=jnp.float32)
                 + b_ref[...])
    rid = i * _BM + lax.broadcasted_iota(jnp.int32, (_BM, 1), 0)
    t = jnp.where(rid == _PAD, 0.0, t)
    z = jnp.zeros((_BM, _CW - 80), jnp.float32)
    o_ref[...] = jnp.concatenate([t, itm_ref[...], z], axis=1)


def _build_table(genome, item_table, w_t, bias):
    m, k = genome.shape
    grid = (m + _BM - 1) // _BM
    return pl.pallas_call(
        _proj_body,
        grid=(grid,),
        in_specs=[
            pl.BlockSpec((_BM, k), lambda i: (i, 0)),
            pl.BlockSpec((_BM, 40), lambda i: (i, 0)),
            pl.BlockSpec((k, 40), lambda i: (0, 0)),
            pl.BlockSpec((1, 40), lambda i: (0, 0)),
        ],
        out_specs=pl.BlockSpec((_BM, _CW), lambda i: (i, 0)),
        out_shape=jax.ShapeDtypeStruct((m, _CW), jnp.float32),
    )(genome, item_table, w_t, bias)


def _sc_body(idx_hbm, tgt_hbm, rat_hbm, tab_hbm, outp_hbm, outt_hbm,
             idx_v, wv_v, row_a, row_b, out_v, tgt_v, trow_v, sema, semb):
    c = lax.axis_index("c")
    s = lax.axis_index("s")
    w = s * _NC + c
    pltpu.sync_copy(idx_hbm.at[w], idx_v)
    pltpu.sync_copy(rat_hbm.at[w], wv_v)

    def pool(j, buf):
        for u in range(2):
            def hloop(h, accs):
                wvec = wv_v[pl.ds((j * _CH + u * _H + h) * 16, 16)]
                row = u * _H + h
                return tuple(accs[ss] + wvec * buf[row, pl.ds(16 * ss, 16)]
                             for ss in range(8))

            accs = lax.fori_loop(
                0, _H, hloop,
                tuple(jnp.zeros((16,), jnp.float32) for _ in range(8)))
            for ss in range(8):
                out_v[j * 2 + u, pl.ds(16 * ss, 16)] = accs[ss]

    pltpu.async_copy(tab_hbm.at[idx_v.at[0]], row_a, sema)
    pltpu.async_copy(tab_hbm.at[idx_v.at[1]], row_b, semb)

    def pair(p, carry):
        ca = 2 * p
        cb = 2 * p + 1
        pltpu.make_async_copy(tab_hbm.at[idx_v.at[ca]], row_a, sema).wait()
        pool(ca, row_a)

        @pl.when(p < _NCH // 2 - 1)
        def _():
            pltpu.async_copy(tab_hbm.at[idx_v.at[ca + 2]], row_a, sema)

        pltpu.make_async_copy(tab_hbm.at[idx_v.at[cb]], row_b, semb).wait()
        pool(cb, row_b)

        @pl.when(p < _NCH // 2 - 1)
        def _():
            pltpu.async_copy(tab_hbm.at[idx_v.at[cb + 2]], row_b, semb)

        return carry

    lax.fori_loop(0, _NCH // 2, pair, 0)
    pltpu.sync_copy(out_v, outp_hbm.at[pl.ds(w * _UPW, _UPW)])

    pltpu.sync_copy(tgt_hbm.at[w], tgt_v)
    pltpu.async_copy(tab_hbm.at[tgt_v], trow_v, sema).wait()
    pltpu.sync_copy(trow_v, outt_hbm.at[pl.ds(w * _UPW, _UPW)])


def _sc_gather_pool(idx3, tgt2, rat2, tab):
    fn = functools.partial(
        pl.kernel,
        out_type=[
            jax.ShapeDtypeStruct((_B, _CW), jnp.float32),
            jax.ShapeDtypeStruct((_B, _CW), jnp.float32),
        ],
        mesh=plsc.VectorSubcoreMesh(core_axis_name="c", subcore_axis_name="s"),
        scratch_types=[
            pltpu.VMEM((_NCH, _CH), jnp.int32),
            pltpu.VMEM((_PERW * 16,), jnp.float32),
            pltpu.VMEM((_CH, _CW), jnp.float32),
            pltpu.VMEM((_CH, _CW), jnp.float32),
            pltpu.VMEM((_UPW, _CW), jnp.float32),
            pltpu.VMEM((_UPW,), jnp.int32),
            pltpu.VMEM((_UPW, _CW), jnp.float32),
            pltpu.SemaphoreType.DMA,
            pltpu.SemaphoreType.DMA,
        ],
    )(_sc_body)
    return fn(idx3, tgt2, rat2, tab)


_BB = 128


def _combine_body(p_ref, hidx_ref, rat_ref, ugc_ref, ts_ref, yr_ref,
                  mg_ref, mt_ref, mgt_ref, tg_ref,
                  wug_ref, bug_ref, tst_ref, wts_ref, bts_ref,
                  yrt_ref, wyr_ref, byr_ref,
                  wig_ref, big_ref, wit_ref, bit_ref,
                  wgn_ref, bgn_ref, wie_ref, bie_ref, o_ref):
    f32 = jnp.float32
    wgt = rat_ref[...] * (hidx_ref[...] != _PAD).astype(f32)
    ws = jnp.clip(jnp.sum(jnp.abs(wgt), axis=1, keepdims=True), 1e-6, None)
    pooled = p_ref[...] / ws
    gen = pooled[:, :35]
    hist = pooled[:, 40:80]

    dot = lambda a, b: jnp.dot(a, b, preferred_element_type=f32)
    genre = jnp.tanh(dot(ugc_ref[...], wug_ref[...]) + bug_ref[...])
    ts_oh = (lax.broadcasted_iota(jnp.int32, (_BB, 100), 1) == ts_ref[...]).astype(f32)
    tse = jnp.tanh(dot(dot(ts_oh, tst_ref[...]), wts_ref[...]) + bts_ref[...])
    yr_oh = (lax.broadcasted_iota(jnp.int32, (_BB, 120), 1) == yr_ref[...]).astype(f32)
    yre = jnp.tanh(dot(dot(yr_oh, yrt_ref[...]), wyr_ref[...]) + byr_ref[...])

    ig = jnp.tanh(dot(mg_ref[...], wig_ref[...]) + big_ref[...])
    it = jnp.tanh(dot(mt_ref[...], wit_ref[...]) + bit_ref[...])
    ign = jnp.tanh(dot(mgt_ref[...], wgn_ref[...]) + bgn_ref[...])
    ie = jnp.tanh(dot(tg_ref[...][:, 40:80], wie_ref[...]) + bie_ref[...])

    u = jnp.concatenate([hist, gen, genre, tse], axis=1)
    v = jnp.concatenate([ig, it, ign, ie, yre], axis=1)
    o_ref[...] = jnp.sum(u * v, axis=1, keepdims=True)


def _combine(pooled, hidx, rat, ugc, ts2, yr2, mg, mt, mgt, tgt_rows, consts):
    grid = _B // _BB
    row = lambda i: (i, 0)
    rep = lambda i: (0, 0)
    in_specs = [
        pl.BlockSpec((_BB, _CW), row),
        pl.BlockSpec((_BB, _H), row),
        pl.BlockSpec((_BB, _H), row),
        pl.BlockSpec((_BB, 20), row),
        pl.BlockSpec((_BB, 1), row),
        pl.BlockSpec((_BB, 1), row),
        pl.BlockSpec((_BB, 20), row),
        pl.BlockSpec((_BB, 1000), row),
        pl.BlockSpec((_BB, 1128), row),
        pl.BlockSpec((_BB, _CW), row),
    ] + [pl.BlockSpec(c.shape, rep) for c in consts]
    return pl.pallas_call(
        _combine_body,
        grid=(grid,),
        in_specs=in_specs,
        out_specs=pl.BlockSpec((_BB, 1), row),
        out_shape=jax.ShapeDtypeStruct((_B, 1), jnp.float32),
    )(pooled, hidx, rat, ugc, ts2, yr2, mg, mt, mgt, tgt_rows, *consts)


def kernel(user_genre_contexts, user_watch_history, user_watch_history_ratings,
           timestamps, movie_genres, movie_tags, movie_genome_tags, years,
           target_movieId, genome_context_buffer, item_table, Wie, bie, Wig, big,
           Wit, bit, Wgn, bgn, year_table, Wyr, byr, Wug, bug, ts_table, Wts, bts):
    f32 = jnp.float32
    wgn_t_pad = jnp.zeros((Wgn.shape[1], 40), f32).at[:, :35].set(Wgn.T)
    bgn_pad = jnp.zeros((1, 40), f32).at[0, :35].set(bgn)
    tab = _build_table(genome_context_buffer, item_table, wgn_t_pad, bgn_pad)

    idx = user_watch_history.astype(jnp.int32)
    idx3 = idx.reshape(_NW, _NCH, _CH)
    tgt2 = target_movieId.astype(jnp.int32).reshape(_NW, _UPW)
    rat2 = jnp.broadcast_to(user_watch_history_ratings.reshape(_B * _H, 1),
                            (_B * _H, 16)).reshape(_NW, _PERW * 16)
    pooled, tgtg = _sc_gather_pool(idx3, tgt2, rat2, tab)

    consts = [
        Wug.T, bug.reshape(1, -1), ts_table, Wts.T, bts.reshape(1, -1),
        year_table, Wyr.T, byr.reshape(1, -1),
        Wig.T, big.reshape(1, -1), Wit.T, bit.reshape(1, -1),
        Wgn.T, bgn.reshape(1, -1), Wie.T, bie.reshape(1, -1),
    ]
    out = _combine(pooled, idx, user_watch_history_ratings, user_genre_contexts,
                   timestamps.astype(jnp.int32).reshape(_B, 1),
                   years.astype(jnp.int32).reshape(_B, 1),
                   movie_genres, movie_tags, movie_genome_tags, tgtg, consts)
    return out.reshape(_B)
